# baseline (speedup 1.0000x reference)
import jax
import jax.numpy as jnp
from jax import lax
from jax.experimental import pallas as pl
from jax.experimental.pallas import tpu as pltpu
from jax.experimental.pallas import tpu_sc as plsc

D, H, W, C = 2, 200, 176, 128
NNZ = 60000
R = D * H * W
NC, NS = 2, 16

CH = 8192
CHB = 13
NCHUNK = 9
KPC = 5
NTRASH = 64
SPAN = 3752
SPAN_LAST = NNZ - (NS - 1) * SPAN
NG = 235
BCAP = 4480
ZR = (CH + NTRASH) // NS
WB2 = CH // NS
LWB = (R - (NCHUNK - 1) * CH) // NS


def _sc_body(values, d_hbm, h_hbm, w_hbm, z_hbm, out,
             vals_v, di_v, hi_v, wi_v, bin_v, dst_x, src_x, spm):
    cid = lax.axis_index("c")
    tid = lax.axis_index("s")
    iota = lax.iota(jnp.int32, 16)
    base_row = tid * SPAN
    span = jnp.where(tid == NS - 1, jnp.int32(SPAN_LAST), jnp.int32(SPAN))

    @pl.when(tid < NS - 1)
    def _():
        pltpu.sync_copy(d_hbm.at[pl.ds(base_row, SPAN)], di_v.at[pl.ds(0, SPAN)])
        pltpu.sync_copy(h_hbm.at[pl.ds(base_row, SPAN)], hi_v.at[pl.ds(0, SPAN)])
        pltpu.sync_copy(w_hbm.at[pl.ds(base_row, SPAN)], wi_v.at[pl.ds(0, SPAN)])

    @pl.when(tid == NS - 1)
    def _():
        pltpu.sync_copy(d_hbm.at[pl.ds(base_row, SPAN_LAST)],
                        di_v.at[pl.ds(0, SPAN_LAST)])
        pltpu.sync_copy(h_hbm.at[pl.ds(base_row, SPAN_LAST)],
                        hi_v.at[pl.ds(0, SPAN_LAST)])
        pltpu.sync_copy(w_hbm.at[pl.ds(base_row, SPAN_LAST)],
                        wi_v.at[pl.ds(0, SPAN_LAST)])

    def coords(g):
        o = g * 16
        dv = di_v[pl.ds(o, 16)]
        hv = hi_v[pl.ds(o, 16)]
        wv = wi_v[pl.ds(o, 16)]
        lin = (dv * H + hv) * W + wv
        valid = (o + iota) < span
        kv = lax.shift_right_logical(lin, CHB)
        mine = ((kv & 1) == cid) & valid
        kkv = lax.shift_right_logical(kv, 1)
        return lin, kkv, mine

    def scan_a(g, hist):
        _, kkv, mine = coords(g)
        for kk in range(KPC):
            m = mine & (kkv == kk)
            pc = plsc.all_reduce_population_count(m)
            hist = hist + jnp.where(iota == kk, pc, 0)
        return hist

    hist = lax.fori_loop(0, NG, scan_a, jnp.zeros((16,), jnp.int32))
    rounded = (hist + 127) & (-128)
    bases_v = plsc.cumsum(rounded) - rounded
    counts = [jnp.sum(jnp.where(iota == kk, hist, 0)) for kk in range(KPC)]
    bases = [jnp.sum(jnp.where(iota == kk, bases_v, 0)) for kk in range(KPC)]

    def scan_b(g, curs):
        lin, kkv, mine = coords(g)
        src = base_row + g * 16 + iota
        entry = lax.shift_left(src, CHB) | (lin & (CH - 1))
        pos = jnp.zeros((16,), jnp.int32)
        new_curs = []
        for kk in range(KPC):
            m = mine & (kkv == kk)
            rank = plsc.cumsum(jnp.where(m, 1, 0))
            pos = jnp.where(m, curs[kk] + rank - 1, pos)
            new_curs.append(curs[kk] + plsc.all_reduce_population_count(m))
        plsc.store_scatter(bin_v, [pos], entry, mask=mine)
        return tuple(new_curs)

    lax.fori_loop(0, NG, scan_b,
                  tuple(jnp.full((16,), 0, jnp.int32) + b for b in bases))

    for kk in range(KPC):
        k = kk * 2 + cid

        @pl.when(k < NCHUNK)
        def _(kk=kk, k=k):
            pltpu.sync_copy(z_hbm, spm.at[pl.ds(tid * ZR, ZR)])
            plsc.subcore_barrier()
            n = counts[kk]
            b0 = bases[kk]
            npc = lax.shift_right_logical(n + 127, 7)

            def piece(pi, _):
                pb = b0 + pi * 128
                for g in range(8):
                    e = bin_v[pl.ds(pb + g * 16, 16)]
                    gpos = pi * 128 + g * 16 + iota
                    okv = gpos < n
                    dst = jnp.where(okv, e & (CH - 1),
                                    CH + ((g * 16 + iota) & (NTRASH - 1)))
                    sr = jnp.where(okv, lax.shift_right_logical(e, CHB), 0)
                    dst_x[pl.ds(g * 16, 16)] = dst
                    src_x[pl.ds(g * 16, 16)] = sr
                pltpu.sync_copy(values.at[src_x], vals_v)
                pltpu.sync_copy(vals_v, spm.at[dst_x], add=True)
                return 0

            lax.fori_loop(0, npc, piece, 0)
            plsc.subcore_barrier()

            @pl.when(k == NCHUNK - 1)
            def _():
                pltpu.sync_copy(spm.at[pl.ds(tid * LWB, LWB)],
                                out.at[pl.ds(k * CH + tid * LWB, LWB)])

            @pl.when(k < NCHUNK - 1)
            def _():
                pltpu.sync_copy(spm.at[pl.ds(tid * WB2, WB2)],
                                out.at[pl.ds(k * CH + tid * WB2, WB2)])
            plsc.subcore_barrier()


def _sc_scatter(values, d_i, h_i, w_i):
    zeros = jnp.zeros((ZR, C), jnp.float32)
    mesh = plsc.VectorSubcoreMesh(core_axis_name="c", subcore_axis_name="s")
    return pl.kernel(
        _sc_body,
        out_type=jax.ShapeDtypeStruct((R, C), jnp.float32),
        mesh=mesh,
        compiler_params=pltpu.CompilerParams(needs_layout_passes=False),
        scratch_types=[
            pltpu.VMEM((128, C), jnp.float32),
            pltpu.VMEM((SPAN + 8, ), jnp.int32),
            pltpu.VMEM((SPAN + 8, ), jnp.int32),
            pltpu.VMEM((SPAN + 8, ), jnp.int32),
            pltpu.VMEM((BCAP,), jnp.int32),
            pltpu.VMEM((128,), jnp.int32),
            pltpu.VMEM((128,), jnp.int32),
            pltpu.VMEM_SHARED((CH + NTRASH, C), jnp.float32),
        ],
    )(values, d_i, h_i, w_i, zeros)


RB = 440

import numpy as _np
_PE = _np.zeros((C, C * D), _np.float32)
_PO = _np.zeros((C, C * D), _np.float32)
_PE[_np.arange(C), 2 * _np.arange(C)] = 1.0
_PO[_np.arange(C), 2 * _np.arange(C) + 1] = 1.0


def _il_body(x_ref, pe_ref, po_ref, o_ref):
    o_ref[...] = (
        jnp.dot(x_ref[0], pe_ref[...], preferred_element_type=jnp.float32)
        + jnp.dot(x_ref[1], po_ref[...], preferred_element_type=jnp.float32))


def _tc_interleave(dense):
    return pl.pallas_call(
        _il_body,
        grid=((H * W) // RB,),
        in_specs=[
            pl.BlockSpec((D, RB, C), lambda i: (0, i, 0)),
            pl.BlockSpec((C, C * D), lambda i: (0, 0)),
            pl.BlockSpec((C, C * D), lambda i: (0, 0)),
        ],
        out_specs=pl.BlockSpec((RB, C * D), lambda i: (i, 0)),
        out_shape=jax.ShapeDtypeStruct((H * W, C * D), jnp.float32),
    )(dense, jnp.asarray(_PE), jnp.asarray(_PO))


@jax.jit
def kernel(values, indices_d, indices_h, indices_w):
    values = values.astype(jnp.float32)
    d_i = indices_d.astype(jnp.int32)
    h_i = indices_h.astype(jnp.int32)
    w_i = indices_w.astype(jnp.int32)
    dense = _sc_scatter(values, d_i, h_i, w_i)
    out = _tc_interleave(dense.reshape(D, H * W, C))
    return jnp.transpose(out.reshape(H, W, C * D), (2, 0, 1))[None]


# speedup vs baseline: 2.3150x; 2.3150x over previous
import jax
import jax.numpy as jnp
from jax import lax
from jax.experimental import pallas as pl
from jax.experimental.pallas import tpu as pltpu
from jax.experimental.pallas import tpu_sc as plsc

D, H, W, C = 2, 200, 176, 128
NNZ = 60000
R = D * H * W
NC, NS = 2, 16

PASSES = 3
CHUNK = 11776
NTRASH = 64
BB = 128
NBATCH = 469
SHORT = NNZ - BB * (NBATCH - 1)
NBI = 30
ZROWS = (CHUNK + NTRASH) // NS
WB = CHUNK // NS
LASTWB = R - (NC * PASSES - 1) * CHUNK - (NS - 1) * WB


def _sc_body(values, d_hbm, h_hbm, w_hbm, z_hbm, out,
             vals_v0, vals_v1, di_v0, di_v1, hi_v0, hi_v1, wi_v0, wi_v1,
             idx0, idx1, sem0, sem1, ssem0, ssem1, spm):
    vals_b = [vals_v0, vals_v1]
    di_b = [di_v0, di_v1]
    hi_b = [hi_v0, hi_v1]
    wi_b = [wi_v0, wi_v1]
    idx_b = [idx0, idx1]
    sem_b = [sem0, sem1]
    ssem_b = [ssem0, ssem1]
    cid = lax.axis_index("c")
    tid = lax.axis_index("s")
    iota = lax.iota(jnp.int32, 16)

    def start_stage(i, u):
        off = (tid + NS * i) * BB
        return [
            pltpu.async_copy(values.at[pl.ds(off, BB)], vals_b[u], sem_b[u]),
            pltpu.async_copy(d_hbm.at[pl.ds(off, BB)], di_b[u], sem_b[u]),
            pltpu.async_copy(h_hbm.at[pl.ds(off, BB)], hi_b[u], sem_b[u]),
            pltpu.async_copy(w_hbm.at[pl.ds(off, BB)], wi_b[u], sem_b[u]),
        ]

    def stage_sync(off, nrows, u):
        pltpu.sync_copy(values.at[pl.ds(off, nrows)],
                        vals_b[u].at[pl.ds(0, nrows)])
        pltpu.sync_copy(d_hbm.at[pl.ds(off, nrows)], di_b[u].at[pl.ds(0, nrows)])
        pltpu.sync_copy(h_hbm.at[pl.ds(off, nrows)], hi_b[u].at[pl.ds(0, nrows)])
        pltpu.sync_copy(w_hbm.at[pl.ds(off, nrows)], wi_b[u].at[pl.ds(0, nrows)])

    def compute_idx(u, lo, limit):
        def grp(gg, _):
            o = gg * 16
            dv = di_b[u][pl.ds(o, 16)]
            hv = hi_b[u][pl.ds(o, 16)]
            wv = wi_b[u][pl.ds(o, 16)]
            lin = dv * (H * W) + hv * W + wv
            ok = (lin >= lo) & (lin < lo + CHUNK) & ((o + iota) < limit)
            local = jnp.where(ok, lin - lo, CHUNK + (lin & (NTRASH - 1)))
            idx_b[u][pl.ds(o, 16)] = local
            return 0

        lax.fori_loop(0, BB // 16, grp, 0)

    def process(u, lo, limit):
        compute_idx(u, lo, limit)
        pltpu.sync_copy(vals_b[u], spm.at[idx_b[u]], add=True)

    for p in range(PASSES):
        k = cid * PASSES + p
        lo = k * CHUNK
        pltpu.sync_copy(z_hbm, spm.at[pl.ds(tid * ZROWS, ZROWS)])
        plsc.subcore_barrier()

        descs = start_stage(0, 0)
        scat = [None, None]
        for i in range(NBI - 1):
            u = i % 2
            nxt = None
            if i + 1 < NBI - 1:
                if scat[1 - u] is not None:
                    scat[1 - u].wait()
                nxt = start_stage(i + 1, 1 - u)
            for dsc in descs:
                dsc.wait()
            compute_idx(u, lo, jnp.int32(BB))
            scat[u] = pltpu.async_copy(vals_b[u], spm.at[idx_b[u]],
                                       ssem_b[u], add=True)
            descs = nxt
        for s in scat:
            if s is not None:
                s.wait()
        b = tid + NS * (NBI - 1)
        u = (NBI - 1) % 2

        @pl.when(b < NBATCH - 1)
        def _():
            stage_sync(b * BB, BB, u)

        @pl.when(b == NBATCH - 1)
        def _():
            stage_sync(b * BB, SHORT, u)
        limit = jnp.where(
            b == NBATCH - 1, jnp.int32(SHORT),
            jnp.where(b < NBATCH - 1, jnp.int32(BB), jnp.int32(0)))
        process(u, lo, limit)
        plsc.subcore_barrier()

        partial = (k == NC * PASSES - 1) & (tid == NS - 1)

        @pl.when(partial)
        def _():
            pltpu.sync_copy(spm.at[pl.ds(tid * WB, LASTWB)],
                            out.at[pl.ds(lo + tid * WB, LASTWB)])

        @pl.when(~partial)
        def _():
            pltpu.sync_copy(spm.at[pl.ds(tid * WB, WB)],
                            out.at[pl.ds(lo + tid * WB, WB)])
        plsc.subcore_barrier()


def _sc_scatter(values, d_i, h_i, w_i):
    zeros = jnp.zeros((ZROWS, C), jnp.float32)
    mesh = plsc.VectorSubcoreMesh(core_axis_name="c", subcore_axis_name="s")
    return pl.kernel(
        _sc_body,
        out_type=jax.ShapeDtypeStruct((R, C), jnp.float32),
        mesh=mesh,
        scratch_types=[
            pltpu.VMEM((BB, C), jnp.float32),
            pltpu.VMEM((BB, C), jnp.float32),
            pltpu.VMEM((BB,), jnp.int32),
            pltpu.VMEM((BB,), jnp.int32),
            pltpu.VMEM((BB,), jnp.int32),
            pltpu.VMEM((BB,), jnp.int32),
            pltpu.VMEM((BB,), jnp.int32),
            pltpu.VMEM((BB,), jnp.int32),
            pltpu.VMEM((128,), jnp.int32),
            pltpu.VMEM((128,), jnp.int32),
            pltpu.SemaphoreType.DMA,
            pltpu.SemaphoreType.DMA,
            pltpu.SemaphoreType.DMA,
            pltpu.SemaphoreType.DMA,
            pltpu.VMEM_SHARED((CHUNK + NTRASH, C), jnp.float32),
        ],
    )(values, d_i, h_i, w_i, zeros)


RB = 440

import numpy as _np
_PE = _np.zeros((C, C * D), _np.float32)
_PO = _np.zeros((C, C * D), _np.float32)
_PE[_np.arange(C), 2 * _np.arange(C)] = 1.0
_PO[_np.arange(C), 2 * _np.arange(C) + 1] = 1.0


def _il_body(x_ref, pe_ref, po_ref, o_ref):
    o_ref[...] = (
        jnp.dot(x_ref[0], pe_ref[...], preferred_element_type=jnp.float32)
        + jnp.dot(x_ref[1], po_ref[...], preferred_element_type=jnp.float32))


def _tc_interleave(dense):
    return pl.pallas_call(
        _il_body,
        grid=((H * W) // RB,),
        in_specs=[
            pl.BlockSpec((D, RB, C), lambda i: (0, i, 0)),
            pl.BlockSpec((C, C * D), lambda i: (0, 0)),
            pl.BlockSpec((C, C * D), lambda i: (0, 0)),
        ],
        out_specs=pl.BlockSpec((RB, C * D), lambda i: (i, 0)),
        out_shape=jax.ShapeDtypeStruct((H * W, C * D), jnp.float32),
    )(dense, jnp.asarray(_PE), jnp.asarray(_PO))


@jax.jit
def kernel(values, indices_d, indices_h, indices_w):
    values = values.astype(jnp.float32)
    d_i = indices_d.astype(jnp.int32)
    h_i = indices_h.astype(jnp.int32)
    w_i = indices_w.astype(jnp.int32)
    dense = _sc_scatter(values, d_i, h_i, w_i)
    out = _tc_interleave(dense.reshape(D, H * W, C))
    return jnp.transpose(out.reshape(H, W, C * D), (2, 0, 1))[None]


# speedup vs baseline: 2.6602x; 1.1491x over previous
import jax
import jax.numpy as jnp
from jax import lax
from jax.experimental import pallas as pl
from jax.experimental.pallas import tpu as pltpu
from jax.experimental.pallas import tpu_sc as plsc

D, H, W, C = 2, 200, 176, 128
NNZ = 60000
R = D * H * W
NC, NS = 2, 16

PASSES = 3
CHUNK = 11776
NTRASH = 64
BB = 128
NBATCH = 469
SHORT = NNZ - BB * (NBATCH - 1)
NBI = 30
ZROWS = (CHUNK + NTRASH) // NS
WB = CHUNK // NS
LASTWB = R - (NC * PASSES - 1) * CHUNK - (NS - 1) * WB


def _sc_body(values, d_hbm, h_hbm, w_hbm, z_hbm, out,
             vals_v0, vals_v1, di_v0, di_v1, hi_v0, hi_v1, wi_v0, wi_v1,
             idx0, idx1, sem0, sem1, ssem0, ssem1, spm):
    vals_b = [vals_v0, vals_v1]
    di_b = [di_v0, di_v1]
    hi_b = [hi_v0, hi_v1]
    wi_b = [wi_v0, wi_v1]
    idx_b = [idx0, idx1]
    sem_b = [sem0, sem1]
    ssem_b = [ssem0, ssem1]
    cid = lax.axis_index("c")
    tid = lax.axis_index("s")
    iota = lax.iota(jnp.int32, 16)

    def start_stage(i, u):
        off = (tid + NS * i) * BB
        return [
            pltpu.async_copy(values.at[pl.ds(off, BB)], vals_b[u], sem_b[u]),
            pltpu.async_copy(d_hbm.at[pl.ds(off, BB)], di_b[u], sem_b[u]),
            pltpu.async_copy(h_hbm.at[pl.ds(off, BB)], hi_b[u], sem_b[u]),
            pltpu.async_copy(w_hbm.at[pl.ds(off, BB)], wi_b[u], sem_b[u]),
        ]

    def stage_sync(off, nrows, u):
        pltpu.sync_copy(values.at[pl.ds(off, nrows)],
                        vals_b[u].at[pl.ds(0, nrows)])
        pltpu.sync_copy(d_hbm.at[pl.ds(off, nrows)], di_b[u].at[pl.ds(0, nrows)])
        pltpu.sync_copy(h_hbm.at[pl.ds(off, nrows)], hi_b[u].at[pl.ds(0, nrows)])
        pltpu.sync_copy(w_hbm.at[pl.ds(off, nrows)], wi_b[u].at[pl.ds(0, nrows)])

    def compute_idx(u, lo, limit):
        def grp(gg, _):
            o = gg * 16
            dv = di_b[u][pl.ds(o, 16)]
            hv = hi_b[u][pl.ds(o, 16)]
            wv = wi_b[u][pl.ds(o, 16)]
            lin = dv * (H * W) + hv * W + wv
            ok = (lin >= lo) & (lin < lo + CHUNK) & ((o + iota) < limit)
            local = jnp.where(ok, lin - lo, CHUNK + (lin & (NTRASH - 1)))
            idx_b[u][pl.ds(o, 16)] = local
            return 0

        lax.fori_loop(0, BB // 16, grp, 0)

    def process(u, lo, limit):
        compute_idx(u, lo, limit)
        pltpu.sync_copy(vals_b[u], spm.at[idx_b[u]], add=True)

    for p in range(PASSES):
        k = cid * PASSES + p
        lo = k * CHUNK
        pltpu.sync_copy(z_hbm, spm.at[pl.ds(tid * ZROWS, ZROWS)])
        plsc.subcore_barrier()

        descs = start_stage(0, 0)
        scat = [None, None]
        for i in range(NBI - 1):
            u = i % 2
            nxt = None
            if i + 1 < NBI - 1:
                if scat[1 - u] is not None:
                    scat[1 - u].wait()
                nxt = start_stage(i + 1, 1 - u)
            for dsc in descs:
                dsc.wait()
            compute_idx(u, lo, jnp.int32(BB))
            scat[u] = pltpu.async_copy(vals_b[u], spm.at[idx_b[u]],
                                       ssem_b[u], add=True)
            descs = nxt
        for s in scat:
            if s is not None:
                s.wait()
        b = tid + NS * (NBI - 1)
        u = (NBI - 1) % 2

        @pl.when(b < NBATCH - 1)
        def _():
            stage_sync(b * BB, BB, u)

        @pl.when(b == NBATCH - 1)
        def _():
            stage_sync(b * BB, SHORT, u)
        limit = jnp.where(
            b == NBATCH - 1, jnp.int32(SHORT),
            jnp.where(b < NBATCH - 1, jnp.int32(BB), jnp.int32(0)))
        process(u, lo, limit)
        plsc.subcore_barrier()

        partial = (k == NC * PASSES - 1) & (tid == NS - 1)

        @pl.when(partial)
        def _():
            pltpu.sync_copy(spm.at[pl.ds(tid * WB, LASTWB)],
                            out.at[pl.ds(lo + tid * WB, LASTWB)])

        @pl.when(~partial)
        def _():
            pltpu.sync_copy(spm.at[pl.ds(tid * WB, WB)],
                            out.at[pl.ds(lo + tid * WB, WB)])
        plsc.subcore_barrier()


def _sc_scatter(values, d_i, h_i, w_i):
    zeros = jnp.zeros((ZROWS, C), jnp.float32)
    mesh = plsc.VectorSubcoreMesh(core_axis_name="c", subcore_axis_name="s")
    return pl.kernel(
        _sc_body,
        out_type=jax.ShapeDtypeStruct((R, C), jnp.float32),
        mesh=mesh,
        scratch_types=[
            pltpu.VMEM((BB, C), jnp.float32),
            pltpu.VMEM((BB, C), jnp.float32),
            pltpu.VMEM((BB,), jnp.int32),
            pltpu.VMEM((BB,), jnp.int32),
            pltpu.VMEM((BB,), jnp.int32),
            pltpu.VMEM((BB,), jnp.int32),
            pltpu.VMEM((BB,), jnp.int32),
            pltpu.VMEM((BB,), jnp.int32),
            pltpu.VMEM((128,), jnp.int32),
            pltpu.VMEM((128,), jnp.int32),
            pltpu.SemaphoreType.DMA,
            pltpu.SemaphoreType.DMA,
            pltpu.SemaphoreType.DMA,
            pltpu.SemaphoreType.DMA,
            pltpu.VMEM_SHARED((CHUNK + NTRASH, C), jnp.float32),
        ],
    )(values, d_i, h_i, w_i, zeros)


RB = 1760

import numpy as _np
_PE = _np.zeros((C, C * D), _np.float32)
_PO = _np.zeros((C, C * D), _np.float32)
_PE[_np.arange(C), 2 * _np.arange(C)] = 1.0
_PO[_np.arange(C), 2 * _np.arange(C) + 1] = 1.0


def _il_body(x_ref, pe_ref, po_ref, o_ref):
    o_ref[...] = (
        jnp.dot(x_ref[0], pe_ref[...], preferred_element_type=jnp.float32)
        + jnp.dot(x_ref[1], po_ref[...], preferred_element_type=jnp.float32))


def _tc_interleave(dense):
    return pl.pallas_call(
        _il_body,
        grid=((H * W) // RB,),
        in_specs=[
            pl.BlockSpec((D, RB, C), lambda i: (0, i, 0)),
            pl.BlockSpec((C, C * D), lambda i: (0, 0)),
            pl.BlockSpec((C, C * D), lambda i: (0, 0)),
        ],
        out_specs=pl.BlockSpec((RB, C * D), lambda i: (i, 0)),
        out_shape=jax.ShapeDtypeStruct((H * W, C * D), jnp.float32),
    )(dense, jnp.asarray(_PE), jnp.asarray(_PO))


@jax.jit
def kernel(values, indices_d, indices_h, indices_w):
    values = values.astype(jnp.float32)
    d_i = indices_d.astype(jnp.int32)
    h_i = indices_h.astype(jnp.int32)
    w_i = indices_w.astype(jnp.int32)
    dense = _sc_scatter(values, d_i, h_i, w_i)
    out = _tc_interleave(dense.reshape(D, H * W, C))
    return jnp.transpose(out.reshape(H, W, C * D), (2, 0, 1))[None]


# speedup vs baseline: 2.7074x; 1.0177x over previous
import jax
import jax.numpy as jnp
from jax import lax
from jax.experimental import pallas as pl
from jax.experimental.pallas import tpu as pltpu
from jax.experimental.pallas import tpu_sc as plsc

D, H, W, C = 2, 200, 176, 128
NNZ = 60000
R = D * H * W
NC, NS = 2, 16

PASSES = 3
CHUNK = 11776
NTRASH = 64
BB = 128
NBATCH = 469
SHORT = NNZ - BB * (NBATCH - 1)
NBI = 30
ZROWS = (CHUNK + NTRASH) // NS
WB = CHUNK // NS
LASTWB = R - (NC * PASSES - 1) * CHUNK - (NS - 1) * WB


def _sc_body(values, d_hbm, h_hbm, w_hbm, z_hbm, out,
             vals_v0, vals_v1, di_v0, di_v1, hi_v0, hi_v1, wi_v0, wi_v1,
             idx0, idx1, sem0, sem1, ssem0, ssem1, spm):
    vals_b = [vals_v0, vals_v1]
    di_b = [di_v0, di_v1]
    hi_b = [hi_v0, hi_v1]
    wi_b = [wi_v0, wi_v1]
    idx_b = [idx0, idx1]
    sem_b = [sem0, sem1]
    ssem_b = [ssem0, ssem1]
    cid = lax.axis_index("c")
    tid = lax.axis_index("s")
    iota = lax.iota(jnp.int32, 16)

    def start_stage(i, u):
        off = (tid + NS * i) * BB
        return [
            pltpu.async_copy(values.at[pl.ds(off, BB)], vals_b[u], sem_b[u]),
            pltpu.async_copy(d_hbm.at[pl.ds(off, BB)], di_b[u], sem_b[u]),
            pltpu.async_copy(h_hbm.at[pl.ds(off, BB)], hi_b[u], sem_b[u]),
            pltpu.async_copy(w_hbm.at[pl.ds(off, BB)], wi_b[u], sem_b[u]),
        ]

    def stage_sync(off, nrows, u):
        pltpu.sync_copy(values.at[pl.ds(off, nrows)],
                        vals_b[u].at[pl.ds(0, nrows)])
        pltpu.sync_copy(d_hbm.at[pl.ds(off, nrows)], di_b[u].at[pl.ds(0, nrows)])
        pltpu.sync_copy(h_hbm.at[pl.ds(off, nrows)], hi_b[u].at[pl.ds(0, nrows)])
        pltpu.sync_copy(w_hbm.at[pl.ds(off, nrows)], wi_b[u].at[pl.ds(0, nrows)])

    def compute_idx(u, lo, limit):
        def grp(gg, _):
            o = gg * 16
            dv = di_b[u][pl.ds(o, 16)]
            hv = hi_b[u][pl.ds(o, 16)]
            wv = wi_b[u][pl.ds(o, 16)]
            lin = dv * (H * W) + hv * W + wv
            ok = (lin >= lo) & (lin < lo + CHUNK) & ((o + iota) < limit)
            local = jnp.where(ok, lin - lo, CHUNK + (lin & (NTRASH - 1)))
            idx_b[u][pl.ds(o, 16)] = local
            return 0

        lax.fori_loop(0, BB // 16, grp, 0)

    def process(u, lo, limit):
        compute_idx(u, lo, limit)
        pltpu.sync_copy(vals_b[u], spm.at[idx_b[u]], add=True)

    for p in range(PASSES):
        k = cid * PASSES + p
        lo = k * CHUNK
        pltpu.sync_copy(z_hbm, spm.at[pl.ds(tid * ZROWS, ZROWS)])
        plsc.subcore_barrier()

        descs = start_stage(0, 0)
        scat = [None, None]
        for i in range(NBI - 1):
            u = i % 2
            nxt = None
            if i + 1 < NBI - 1:
                if scat[1 - u] is not None:
                    scat[1 - u].wait()
                nxt = start_stage(i + 1, 1 - u)
            for dsc in descs:
                dsc.wait()
            compute_idx(u, lo, jnp.int32(BB))
            scat[u] = pltpu.async_copy(vals_b[u], spm.at[idx_b[u]],
                                       ssem_b[u], add=True)
            descs = nxt
        for s in scat:
            if s is not None:
                s.wait()
        b = tid + NS * (NBI - 1)
        u = (NBI - 1) % 2

        @pl.when(b < NBATCH - 1)
        def _():
            stage_sync(b * BB, BB, u)

        @pl.when(b == NBATCH - 1)
        def _():
            stage_sync(b * BB, SHORT, u)
        limit = jnp.where(
            b == NBATCH - 1, jnp.int32(SHORT),
            jnp.where(b < NBATCH - 1, jnp.int32(BB), jnp.int32(0)))
        process(u, lo, limit)
        plsc.subcore_barrier()

        partial = (k == NC * PASSES - 1) & (tid == NS - 1)

        @pl.when(partial)
        def _():
            pltpu.sync_copy(spm.at[pl.ds(tid * WB, LASTWB)],
                            out.at[pl.ds(lo + tid * WB, LASTWB)])

        @pl.when(~partial)
        def _():
            pltpu.sync_copy(spm.at[pl.ds(tid * WB, WB)],
                            out.at[pl.ds(lo + tid * WB, WB)])
        plsc.subcore_barrier()


def _sc_scatter(values, d_i, h_i, w_i):
    zeros = jnp.zeros((ZROWS, C), jnp.float32)
    mesh = plsc.VectorSubcoreMesh(core_axis_name="c", subcore_axis_name="s")
    return pl.kernel(
        _sc_body,
        out_type=jax.ShapeDtypeStruct((R, C), jnp.float32),
        mesh=mesh,
        scratch_types=[
            pltpu.VMEM((BB, C), jnp.float32),
            pltpu.VMEM((BB, C), jnp.float32),
            pltpu.VMEM((BB,), jnp.int32),
            pltpu.VMEM((BB,), jnp.int32),
            pltpu.VMEM((BB,), jnp.int32),
            pltpu.VMEM((BB,), jnp.int32),
            pltpu.VMEM((BB,), jnp.int32),
            pltpu.VMEM((BB,), jnp.int32),
            pltpu.VMEM((128,), jnp.int32),
            pltpu.VMEM((128,), jnp.int32),
            pltpu.SemaphoreType.DMA,
            pltpu.SemaphoreType.DMA,
            pltpu.SemaphoreType.DMA,
            pltpu.SemaphoreType.DMA,
            pltpu.VMEM_SHARED((CHUNK + NTRASH, C), jnp.float32),
        ],
    )(values, d_i, h_i, w_i, zeros)


RB = 3520

import numpy as _np
_PE = _np.zeros((C, C * D), _np.float32)
_PO = _np.zeros((C, C * D), _np.float32)
_PE[_np.arange(C), 2 * _np.arange(C)] = 1.0
_PO[_np.arange(C), 2 * _np.arange(C) + 1] = 1.0


def _il_body(x_ref, pe_ref, po_ref, o_ref):
    o_ref[...] = (
        jnp.dot(x_ref[0], pe_ref[...], preferred_element_type=jnp.float32)
        + jnp.dot(x_ref[1], po_ref[...], preferred_element_type=jnp.float32))


def _tc_interleave(dense):
    return pl.pallas_call(
        _il_body,
        grid=((H * W) // RB,),
        in_specs=[
            pl.BlockSpec((D, RB, C), lambda i: (0, i, 0)),
            pl.BlockSpec((C, C * D), lambda i: (0, 0)),
            pl.BlockSpec((C, C * D), lambda i: (0, 0)),
        ],
        out_specs=pl.BlockSpec((RB, C * D), lambda i: (i, 0)),
        out_shape=jax.ShapeDtypeStruct((H * W, C * D), jnp.float32),
    )(dense, jnp.asarray(_PE), jnp.asarray(_PO))


@jax.jit
def kernel(values, indices_d, indices_h, indices_w):
    values = values.astype(jnp.float32)
    d_i = indices_d.astype(jnp.int32)
    h_i = indices_h.astype(jnp.int32)
    w_i = indices_w.astype(jnp.int32)
    dense = _sc_scatter(values, d_i, h_i, w_i)
    out = _tc_interleave(dense.reshape(D, H * W, C))
    return jnp.transpose(out.reshape(H, W, C * D), (2, 0, 1))[None]
